# baseline (speedup 1.0000x reference)
import functools

import jax
import jax.numpy as jnp
from jax import lax
from jax.experimental import pallas as pl
from jax.experimental.pallas import tpu as pltpu
from jax.experimental.pallas import tpu_sc as plsc

B, P, C = 4096, 50, 256
N = B * P
NC, NS = 2, 16
NW = NC * NS
SEG_PER_W = N // NW
CH = 64
NCH = SEG_PER_W // CH

_mesh = plsc.VectorSubcoreMesh(core_axis_name="c", subcore_axis_name="s")


@functools.partial(
    pl.kernel,
    out_type=jax.ShapeDtypeStruct((N * C,), jnp.float32),
    mesh=_mesh,
    compiler_params=pltpu.CompilerParams(needs_layout_passes=False),
    scratch_types=[
        pltpu.VMEM((SEG_PER_W,), jnp.int32),
        pltpu.VMEM((CH * C,), jnp.float32),
    ],
)
def _one_hot_sc(x_hbm, out_hbm, idx_v, buf):
    cid = lax.axis_index("c")
    sid = lax.axis_index("s")
    wid = sid * NC + cid
    base = wid * SEG_PER_W

    pltpu.sync_copy(x_hbm.at[pl.ds(base * 1, SEG_PER_W)], idx_v)

    zeros16 = jnp.zeros((16,), jnp.float32)
    ones16 = jnp.ones((16,), jnp.float32)
    iota16 = lax.iota(jnp.int32, 16)

    def _zero(i, carry):
        buf[pl.ds(i * 16, 16)] = zeros16
        return carry

    lax.fori_loop(0, CH * C // 16, _zero, 0)

    def _scatter(c, val):
        for j in range(CH // 16):
            idx16 = idx_v[pl.ds(c * CH + j * 16, 16)]
            pos = (j * 16 + iota16) * C + idx16
            plsc.store_scatter(buf, [pos], val)

    def _chunk(c, carry):
        _scatter(c, ones16)
        pltpu.sync_copy(buf, out_hbm.at[pl.ds((base + c * CH) * C, CH * C)])
        _scatter(c, zeros16)
        return carry

    lax.fori_loop(0, NCH, _chunk, 0)


def kernel(x):
    out = _one_hot_sc(x.reshape(-1))
    return out.reshape(B, P, C)
